# baseline (speedup 1.0000x reference)
import functools

import jax
import jax.numpy as jnp
from jax import lax
from jax.experimental import pallas as pl
from jax.experimental.pallas import tpu as pltpu
from jax.experimental.pallas import tpu_sc as plsc

_H = 32
_W = 32
_D = 768
_SEQ = _H * _W
_NB = 4
_L = 16
_NC = 2
_DC = _D // _L
_Q = 4
_QW = _W // _Q
_QCH = _QW * _DC


def _pe_sc_body(row_hbm, col_hbm, out_hbm, row_v, col_v, out_v,
                sem_r, sem_c, sem_o):
    wid = lax.axis_index("s") * _NC + lax.axis_index("c")
    h_r = pltpu.async_copy(row_hbm.at[wid], row_v, sem_r)
    h_c = pltpu.async_copy(col_hbm, col_v, sem_c)
    h_r.wait()
    h_c.wait()

    outs = []
    for q in range(_Q):

        @plsc.parallel_loop(q * _QW, (q + 1) * _QW, unroll=2)
        def _(w):
            for ci in range(_DC):
                out_v[w, pl.ds(ci * _L, _L)] = (
                    col_v[w, pl.ds(ci * _L, _L)] + row_v[pl.ds(ci * _L, _L)]
                )

        outs.append(pltpu.async_copy(
            out_v.at[pl.ds(q * _QW, _QW)],
            out_hbm.at[pl.ds(wid * _W + q * _QW, _QW)],
            sem_o,
        ))
    for h in outs:
        h.wait()


_pe_sc = functools.partial(
    pl.kernel,
    out_type=jax.ShapeDtypeStruct((_SEQ, _D), jnp.float32),
    mesh=plsc.VectorSubcoreMesh(core_axis_name="c", subcore_axis_name="s"),
    scratch_types=[
        pltpu.VMEM((_D,), jnp.float32),
        pltpu.VMEM((_W, _D), jnp.float32),
        pltpu.VMEM((_W, _D), jnp.float32),
        pltpu.SemaphoreType.DMA,
        pltpu.SemaphoreType.DMA,
        pltpu.SemaphoreType.DMA,
    ],
)(_pe_sc_body)


def _add_body(x_ref, pe_ref, o_ref):
    o_ref[...] = x_ref[...] + pe_ref[...][None]


def kernel(x, pos_row, pos_col):
    B, SEQ, D = x.shape
    pe = _pe_sc(pos_row, pos_col)
    out = pl.pallas_call(
        _add_body,
        grid=(B // _NB,),
        in_specs=[
            pl.BlockSpec((_NB, SEQ, D), lambda b: (b, 0, 0)),
            pl.BlockSpec((SEQ, D), lambda b: (0, 0)),
        ],
        out_specs=pl.BlockSpec((_NB, SEQ, D), lambda b: (b, 0, 0)),
        out_shape=jax.ShapeDtypeStruct((B, SEQ, D), x.dtype),
    )(x, pe)
    return out


# speedup vs baseline: 1.0455x; 1.0455x over previous
import functools

import jax
import jax.numpy as jnp
from jax import lax
from jax.experimental import pallas as pl
from jax.experimental.pallas import tpu as pltpu
from jax.experimental.pallas import tpu_sc as plsc

_H = 32
_W = 32
_D = 768
_SEQ = _H * _W
_NB = 4
_L = 16
_NC = 2
_DC = _D // _L
_Q = 4
_QW = _W // _Q
_QCH = _QW * _DC


def _pe_sc_body(row_hbm, col_hbm, out_hbm, row_v, col_v, out_v,
                sem_r, sem_c, sem_o):
    wid = lax.axis_index("s") * _NC + lax.axis_index("c")
    h_r = pltpu.async_copy(row_hbm.at[wid], row_v, sem_r)
    h_c = pltpu.async_copy(col_hbm, col_v, sem_c)
    h_r.wait()
    h_c.wait()

    outs = []
    for q in range(_Q):

        @plsc.parallel_loop(q * _QCH, (q + 1) * _QCH, unroll=8)
        def _(i):
            w = i // _DC
            ci = lax.rem(i, _DC)
            out_v[w, pl.ds(ci * _L, _L)] = (
                col_v[w, pl.ds(ci * _L, _L)] + row_v[pl.ds(ci * _L, _L)]
            )

        outs.append(pltpu.async_copy(
            out_v.at[pl.ds(q * _QW, _QW)],
            out_hbm.at[pl.ds(wid * _W + q * _QW, _QW)],
            sem_o,
        ))
    for h in outs:
        h.wait()


_pe_sc = functools.partial(
    pl.kernel,
    out_type=jax.ShapeDtypeStruct((_SEQ, _D), jnp.float32),
    mesh=plsc.VectorSubcoreMesh(core_axis_name="c", subcore_axis_name="s"),
    scratch_types=[
        pltpu.VMEM((_D,), jnp.float32),
        pltpu.VMEM((_W, _D), jnp.float32),
        pltpu.VMEM((_W, _D), jnp.float32),
        pltpu.SemaphoreType.DMA,
        pltpu.SemaphoreType.DMA,
        pltpu.SemaphoreType.DMA,
    ],
)(_pe_sc_body)


def _add_body(x_ref, pe_ref, o_ref):
    o_ref[...] = x_ref[...] + pe_ref[...][None]


def kernel(x, pos_row, pos_col):
    B, SEQ, D = x.shape
    pe = _pe_sc(pos_row, pos_col)
    out = pl.pallas_call(
        _add_body,
        grid=(B // _NB,),
        in_specs=[
            pl.BlockSpec((_NB, SEQ, D), lambda b: (b, 0, 0)),
            pl.BlockSpec((SEQ, D), lambda b: (0, 0)),
        ],
        out_specs=pl.BlockSpec((_NB, SEQ, D), lambda b: (b, 0, 0)),
        out_shape=jax.ShapeDtypeStruct((B, SEQ, D), x.dtype),
    )(x, pe)
    return out
